# Initial kernel scaffold; baseline (speedup 1.0000x reference)
import jax
import jax.numpy as jnp
from jax.experimental import pallas as pl


def kernel(x, edge_index, pre1_W, pre1_b, post1_W, post1_b, lin1_W, lin1_b, bn1_g, bn1_b, pre2_W, pre2_b, post2_W, post2_b, lin2_W, lin2_b, bn2_g, bn2_b):
    raise NotImplementedError("write your pallas kernel here")


# speedup vs baseline: 11.3828x; 11.3828x over previous
import functools
import math

import jax
import jax.numpy as jnp
import numpy as np
from jax.experimental import pallas as pl

N_NODES = 10000
F_IN = 128
TOWERS = 4
G_T = 32
_DEG = np.array([0, 0, 0, 0, 0, 0, 0, 0, 100, 200, 400, 600, 800, 1000, 1200,
                 1300, 1200, 1000, 800, 600, 400, 200, 100, 60, 40],
                dtype=np.float64)
AVG_LOG = float((np.log(np.arange(_DEG.shape[0]) + 1.0) * _DEG).sum() / _DEG.sum())


def _bn_relu_kernel(y_ref, m_ref, v_ref, g_ref, b_ref, o_ref):
    y = y_ref[...]
    m = m_ref[...]
    v = v_ref[...]
    g = g_ref[...]
    b = b_ref[...]
    o_ref[...] = jax.nn.relu((y - m) / jnp.sqrt(v + 1e-5) * g + b)


def _bn_relu(y, g, b):
    m = y.mean(axis=0, keepdims=True)
    v = y.var(axis=0, keepdims=True)
    blk = 1000
    return pl.pallas_call(
        _bn_relu_kernel,
        grid=(N_NODES // blk,),
        in_specs=[
            pl.BlockSpec((blk, F_IN), lambda i: (i, 0)),
            pl.BlockSpec((1, F_IN), lambda i: (0, 0)),
            pl.BlockSpec((1, F_IN), lambda i: (0, 0)),
            pl.BlockSpec((1, F_IN), lambda i: (0, 0)),
            pl.BlockSpec((1, F_IN), lambda i: (0, 0)),
        ],
        out_specs=pl.BlockSpec((blk, F_IN), lambda i: (i, 0)),
        out_shape=jax.ShapeDtypeStruct((N_NODES, F_IN), jnp.float32),
    )(y, m, v, g.reshape(1, -1), b.reshape(1, -1))


def _pna_layer(x, src, dst, cnt, Wp, bp, Wo, bo, Wl, bl):
    n = x.shape[0]
    T, F = TOWERS, F_IN
    Wp_d = Wp[:, :, :F].reshape(T * F, F)
    Wp_s = Wp[:, :, F:].reshape(T * F, F)
    C = jnp.dot(x, Wp_d.T, precision=jax.lax.Precision.HIGHEST) + bp.reshape(1, T * F)
    B = jnp.dot(x, Wp_s.T, precision=jax.lax.Precision.HIGHEST)

    Bg = B[src]
    S = jax.ops.segment_sum(Bg, dst, n)
    Q = jax.ops.segment_sum(Bg * Bg, dst, n)
    MN = jax.ops.segment_min(Bg, dst, n)
    MX = jax.ops.segment_max(Bg, dst, n)

    cntc = jnp.maximum(cnt, 1.0)[:, None]
    has = (cnt > 0)[:, None]
    u = S / cntc
    std = jnp.sqrt(jax.nn.relu(Q / cntc - u * u) + 1e-5)
    mean = jnp.where(has, C + u, 0.0)
    mn = jnp.where(has, C + MN, 0.0)
    mx = jnp.where(has, C + MX, 0.0)

    agg = jnp.concatenate(
        [mean.reshape(n, T, F), mn.reshape(n, T, F),
         mx.reshape(n, T, F), std.reshape(n, T, F)], axis=-1)
    d = jnp.log(cntc + 1.0)[:, :, None]
    Wo_x = Wo[:, :, :F]
    Wo_r = Wo[:, :, F:5 * F]
    Wo_a = Wo[:, :, 5 * F:9 * F]
    Wo_b = Wo[:, :, 9 * F:]
    hi = jax.lax.Precision.HIGHEST
    post = (jnp.einsum('nf,tgf->ntg', x, Wo_x, precision=hi)
            + jnp.einsum('ntf,tgf->ntg', agg, Wo_r, precision=hi)
            + (d / AVG_LOG) * jnp.einsum('ntf,tgf->ntg', agg, Wo_a, precision=hi)
            + (AVG_LOG / d) * jnp.einsum('ntf,tgf->ntg', agg, Wo_b, precision=hi)
            + bo[None])
    y = jnp.dot(post.reshape(n, T * G_T), Wl.T, precision=hi) + bl
    return y


def kernel(x, edge_index, pre1_W, pre1_b, post1_W, post1_b, lin1_W, lin1_b,
           bn1_g, bn1_b, pre2_W, pre2_b, post2_W, post2_b, lin2_W, lin2_b,
           bn2_g, bn2_b):
    src = edge_index[0]
    dst = edge_index[1]
    cnt = jax.ops.segment_sum(jnp.ones_like(src, jnp.float32), dst, N_NODES)
    h = _pna_layer(x, src, dst, cnt, pre1_W, pre1_b, post1_W, post1_b,
                   lin1_W, lin1_b)
    h = _bn_relu(h, bn1_g, bn1_b)
    h = _pna_layer(h, src, dst, cnt, pre2_W, pre2_b, post2_W, post2_b,
                   lin2_W, lin2_b)
    h = _bn_relu(h, bn2_g, bn2_b)
    return h


# speedup vs baseline: 37.9095x; 3.3304x over previous
import functools
import math

import jax
import jax.numpy as jnp
import numpy as np
from jax import lax
from jax.experimental import pallas as pl
from jax.experimental.pallas import tpu as pltpu
from jax.experimental.pallas import tpu_sc as plsc

N_NODES = 10000
N_EDGES = 160000
F_IN = 128
TOWERS = 4
G_T = 32
_DEG = np.array([0, 0, 0, 0, 0, 0, 0, 0, 100, 200, 400, 600, 800, 1000, 1200,
                 1300, 1200, 1000, 800, 600, 400, 200, 100, 60, 40],
                dtype=np.float64)
AVG_LOG = float((np.log(np.arange(_DEG.shape[0]) + 1.0) * _DEG).sum() / _DEG.sum())

NSLICE = 32
EPS = N_EDGES // NSLICE
NRANGES = 64
NRANGE = 160
NPAD = NRANGES * NRANGE
BCAP = 160
BSTRIDE = 176
LCAP = NSLICE * BSTRIDE
KB = 64
NRP = 164
TRASH = 162
HI = jax.lax.Precision.HIGHEST

_mesh = plsc.VectorSubcoreMesh(core_axis_name="c", subcore_axis_name="s")


def _sc_prep(src, dst):
    BLROW = NRANGES * BSTRIDE

    @functools.partial(
        pl.kernel, mesh=_mesh,
        out_type=(jax.ShapeDtypeStruct((NSLICE * BLROW,), jnp.int32),
                  jax.ShapeDtypeStruct(((NRANGES + 1) * 512,), jnp.int32)),
        scratch_types=[pltpu.VMEM((EPS + 16,), jnp.int32),
                       pltpu.VMEM((EPS + 16,), jnp.int32),
                       pltpu.VMEM(((NRANGES + 1) * BSTRIDE,), jnp.int32),
                       pltpu.VMEM(((NRANGES + 1) * 16,), jnp.int32),
                       pltpu.SMEM((NRANGES + 1,), jnp.int32),
                       pltpu.SemaphoreType.DMA],
    )
    def k(src_hbm, dst_hbm, bl_hbm, cnts_hbm, sbuf, dbuf, bkt, cvec, cur, sem):
        w = lax.axis_index("s") * 2 + lax.axis_index("c")
        base = pl.multiple_of(w * EPS, 8)
        pltpu.sync_copy(src_hbm.at[pl.ds(base, EPS)], sbuf.at[pl.ds(0, EPS)])
        pltpu.sync_copy(dst_hbm.at[pl.ds(base, EPS)], dbuf.at[pl.ds(0, EPS)])

        def zc(i, _):
            cur[i] = 0
            return 0
        lax.fori_loop(0, NRANGES + 1, zc, 0)

        recip = np.float32((1.0 / NRANGE) * (1.0 + 2e-6))
        ngroups = (EPS + 15) // 16

        def grp(g, _):
            dv = dbuf[pl.ds(g * 16, 16)]
            sv = sbuf[pl.ds(g * 16, 16)]
            lv = (dv.astype(jnp.float32) * recip).astype(jnp.int32)
            valid = (g * 16 + lax.iota(jnp.int32, 16)) < EPS
            lv = jnp.where(valid, lv, NRANGES)
            pv = sv | ((dv - lv * NRANGE) << 16)
            pv = jnp.where(valid, pv, TRASH << 16)
            for j in range(16):
                l = lv[j]
                p = pv[j]
                c = cur[l]
                bkt[pl.ds(l * BSTRIDE + c, 16)] = jnp.full((16,), p, jnp.int32)
                cur[l] = jnp.minimum(c + 1, BCAP)
            return 0
        lax.fori_loop(0, ngroups, grp, 0)

        trash16 = jnp.full((16,), TRASH << 16, jnp.int32)

        def pad(l, _):
            c = cur[l]
            bkt[pl.ds(l * BSTRIDE + c, 16)] = trash16
            cur[l] = (c + 7) & (-8)
            return 0
        lax.fori_loop(0, NRANGES + 1, pad, 0)

        def cv(i, _):
            cvec[pl.ds(i * 16, 16)] = jnp.full((16,), cur[i], jnp.int32)
            return 0
        lax.fori_loop(0, NRANGES + 1, cv, 0)

        pltpu.sync_copy(bkt.at[pl.ds(0, BLROW)],
                        bl_hbm.at[pl.ds(pl.multiple_of(w * BLROW, 8), BLROW)])
        for l in range(NRANGES + 1):
            coff = pl.multiple_of(l * 512 + w * 16, 8)
            pltpu.async_copy(cvec.at[pl.ds(l * 16, 16)],
                             cnts_hbm.at[pl.ds(coff, 16)], sem)
        for l in range(NRANGES + 1):
            coff = pl.multiple_of(l * 512 + w * 16, 8)
            pltpu.make_async_copy(cvec.at[pl.ds(l * 16, 16)],
                                  cnts_hbm.at[pl.ds(coff, 16)], sem).wait()

    return k(src, dst)


def _make_sc_layer(compute_cnt):
    outs = [jax.ShapeDtypeStruct((TOWERS, NPAD, F_IN), jnp.float32)
            for _ in range(4)]
    if compute_cnt:
        outs.append(jax.ShapeDtypeStruct((NPAD, 16), jnp.float32))

    scratch = [pltpu.VMEM((LCAP,), jnp.int32),
               pltpu.VMEM((LCAP,), jnp.int32),
               pltpu.VMEM((KB, F_IN), jnp.float32),
               pltpu.VMEM((NRP, F_IN), jnp.float32),
               pltpu.VMEM((NRP, F_IN), jnp.float32),
               pltpu.VMEM((NRP, F_IN), jnp.float32),
               pltpu.VMEM((NRP, F_IN), jnp.float32),
               pltpu.VMEM((512,), jnp.int32),
               pltpu.VMEM((NRP, 16), jnp.float32),
               pltpu.SemaphoreType.DMA]

    @functools.partial(pl.kernel, mesh=_mesh, out_type=tuple(outs),
                       scratch_types=scratch)
    def k(bl_hbm, cnts_hbm, tab_hbm, *rest):
        if compute_cnt:
            S_hbm, Q_hbm, MN_hbm, MX_hbm, CNT_hbm = rest[:5]
            scr = rest[5:]
        else:
            S_hbm, Q_hbm, MN_hbm, MX_hbm = rest[:4]
            scr = rest[4:]
        listv, idxv, rows, SU, SQ, MN, MX, ccache, cntacc, sem = scr
        w = lax.axis_index("s") * 2 + lax.axis_index("c")

        zeros16 = jnp.zeros((16,), jnp.float32)
        ones16 = jnp.ones((16,), jnp.float32)
        pinf16 = jnp.full((16,), float("inf"), jnp.float32)
        ninf16 = jnp.full((16,), -float("inf"), jnp.float32)
        trash16 = jnp.full((16,), TRASH << 16, jnp.int32)

        BLROW = NRANGES * BSTRIDE

        def round_body(r, _):
            l = w * 2 + r
            lo = pl.multiple_of(l * NRANGE, 8)
            pltpu.sync_copy(
                cnts_hbm.at[pl.ds(pl.multiple_of(l * 512, 8), 512)], ccache)

            def cp(sl, cur):
                boff = pl.multiple_of(sl * BLROW + l * BSTRIDE, 8)
                pltpu.sync_copy(
                    bl_hbm.at[pl.ds(boff, BSTRIDE)],
                    listv.at[pl.ds(pl.multiple_of(cur, 8), BSTRIDE)])
                crow = ccache[pl.ds(sl * 16, 16)]
                return cur + crow[0]
            cur = lax.fori_loop(0, NSLICE, cp, 0)

            def san(g, _):
                pv = listv[pl.ds(g * 16, 16)]
                vld = (g * 16 + lax.iota(jnp.int32, 16)) < cur
                listv[pl.ds(g * 16, 16)] = jnp.where(vld, pv, trash16)
                return 0
            lax.fori_loop(cur >> 4, LCAP // 16, san, 0)

            nb = (cur + (KB - 1)) >> 6

            def tower(t, _):
                tbase = t * N_NODES

                def zr(i, _):
                    for c8 in range(8):
                        sl16 = pl.ds(c8 * 16, 16)
                        SU[i, sl16] = zeros16
                        SQ[i, sl16] = zeros16
                        MN[i, sl16] = pinf16
                        MX[i, sl16] = ninf16
                    return 0
                lax.fori_loop(0, NRP, zr, 0)

                def ib(g, _):
                    pv = listv[pl.ds(g * 16, 16)]
                    idxv[pl.ds(g * 16, 16)] = (pv & 0xFFFF) + tbase
                    return 0
                lax.fori_loop(0, LCAP // 16, ib, 0)

                def bat(b, _):
                    boff = pl.multiple_of(b * KB, 8)
                    pltpu.async_copy(
                        tab_hbm.at[idxv.at[pl.ds(boff, KB)]], rows, sem
                    ).wait()

                    def sg_body(sg, _):
                        dlv = lax.shift_right_logical(
                            listv[pl.ds(b * KB + sg * 16, 16)], 16)
                        for j in range(16):
                            dl = dlv[j]
                            e = sg * 16 + j
                            for c8 in range(8):
                                sl16 = pl.ds(c8 * 16, 16)
                                bb = rows[e, sl16]
                                plsc.addupdate(SU.at[dl, sl16], bb)
                                plsc.addupdate(SQ.at[dl, sl16], bb * bb)
                                MN[dl, sl16] = jnp.minimum(MN[dl, sl16], bb)
                                MX[dl, sl16] = jnp.maximum(MX[dl, sl16], bb)
                        return 0
                    lax.fori_loop(0, KB // 16, sg_body, 0)
                    return 0
                lax.fori_loop(0, nb, bat, 0)

                pltpu.sync_copy(SU.at[pl.ds(0, NRANGE)],
                                S_hbm.at[t, pl.ds(lo, NRANGE)])
                pltpu.sync_copy(SQ.at[pl.ds(0, NRANGE)],
                                Q_hbm.at[t, pl.ds(lo, NRANGE)])
                pltpu.sync_copy(MN.at[pl.ds(0, NRANGE)],
                                MN_hbm.at[t, pl.ds(lo, NRANGE)])
                pltpu.sync_copy(MX.at[pl.ds(0, NRANGE)],
                                MX_hbm.at[t, pl.ds(lo, NRANGE)])
                return 0
            lax.fori_loop(0, TOWERS, tower, 0)

            if compute_cnt:
                def zc2(i, _):
                    cntacc[i, :] = zeros16
                    return 0
                lax.fori_loop(0, NRP, zc2, 0)

                def cg(g, _):
                    dlv = lax.shift_right_logical(
                        listv[pl.ds(g * 16, 16)], 16)
                    for j in range(16):
                        plsc.addupdate(cntacc.at[dlv[j], :], ones16)
                    return 0
                lax.fori_loop(0, (cur + 15) >> 4, cg, 0)
                pltpu.sync_copy(cntacc.at[pl.ds(0, NRANGE)],
                                CNT_hbm.at[pl.ds(lo, NRANGE)])
            return 0
        lax.fori_loop(0, 2, round_body, 0)

    return k


_sc_layer_cnt = _make_sc_layer(True)
_sc_layer_nocnt = _make_sc_layer(False)


def _pre_kernel(x_ref, wd_ref, ws_ref, b_ref, c_ref, bt_ref):
    xb = x_ref[...]
    c_ref[0] = jnp.dot(xb, wd_ref[...], precision=HI) + b_ref[...]
    bt_ref[...] = jnp.dot(xb, ws_ref[...], precision=HI)


def _pre(x, Wd, Ws, bpf):
    bn, nbk = 1000, N_NODES // 1000
    return pl.pallas_call(
        _pre_kernel,
        grid=(TOWERS, nbk),
        in_specs=[
            pl.BlockSpec((bn, F_IN), lambda t, i: (i, 0)),
            pl.BlockSpec((F_IN, F_IN), lambda t, i: (0, t)),
            pl.BlockSpec((F_IN, F_IN), lambda t, i: (0, t)),
            pl.BlockSpec((1, F_IN), lambda t, i: (0, t)),
        ],
        out_specs=[
            pl.BlockSpec((1, bn, F_IN), lambda t, i: (t, i, 0)),
            pl.BlockSpec((bn, F_IN), lambda t, i: (t * nbk + i, 0)),
        ],
        out_shape=[
            jax.ShapeDtypeStruct((TOWERS, N_NODES, F_IN), jnp.float32),
            jax.ShapeDtypeStruct((TOWERS * N_NODES, F_IN), jnp.float32),
        ],
    )(x, Wd, Ws, bpf)


def _combine_kernel(S_ref, Q_ref, MN_ref, MX_ref, C_ref, cnt_ref, x_ref,
                    Wagg_ref, Wx_ref, Wl_ref, bo_ref, bl_ref, y_ref, st_ref):
    cv = cnt_ref[...][:, 0]
    cntc = jnp.maximum(cv, 1.0)
    has = (cv > 0.0)[:, None]
    xb = x_ref[...]
    base = jnp.dot(xb, Wx_ref[...], precision=HI) + bo_ref[...]
    d = jnp.log(cntc + 1.0)
    s1 = (d / AVG_LOG)[:, None]
    s2 = (AVG_LOG / d)[:, None]
    inv = (1.0 / cntc)[:, None]
    parts = []
    for t in range(TOWERS):
        St = S_ref[t]
        Qt = Q_ref[t]
        Ct = C_ref[t]
        u = St * inv
        std = jnp.sqrt(jax.nn.relu(Qt * inv - u * u) + 1e-5)
        mean = jnp.where(has, Ct + u, 0.0)
        mn = jnp.where(has, Ct + MN_ref[t], 0.0)
        mx = jnp.where(has, Ct + MX_ref[t], 0.0)
        agg = jnp.concatenate([mean, mn, mx, std], axis=1)
        p = jnp.dot(agg, Wagg_ref[t], precision=HI)
        parts.append(p[:, 0:G_T] + s1 * p[:, G_T:2 * G_T]
                     + s2 * p[:, 2 * G_T:3 * G_T])
    post = jnp.concatenate(parts, axis=1) + base
    y = jnp.dot(post, Wl_ref[...], precision=HI) + bl_ref[...]
    y_ref[...] = y

    @pl.when(pl.program_id(0) == 0)
    def _():
        st_ref[...] = jnp.zeros((8, F_IN), jnp.float32)

    bn = y.shape[0]
    rid = (jax.lax.broadcasted_iota(jnp.int32, (bn, 1), 0)
           + pl.program_id(0) * bn)
    ym = jnp.where(rid < N_NODES, y, 0.0)
    upd = jnp.concatenate(
        [jnp.sum(ym, axis=0, keepdims=True),
         jnp.sum(ym * ym, axis=0, keepdims=True),
         jnp.zeros((6, F_IN), jnp.float32)], axis=0)
    st_ref[...] += upd


def _combine(S, Q, MN, MX, C, cnt, x, Wagg, Wxf, WlT, bof, blf):
    bn, nbk = 1280, NPAD // 1280
    full4 = pl.BlockSpec((TOWERS, bn, F_IN), lambda i: (0, i, 0))
    return pl.pallas_call(
        _combine_kernel,
        grid=(nbk,),
        in_specs=[
            full4, full4, full4, full4, full4,
            pl.BlockSpec((bn, 16), lambda i: (i, 0)),
            pl.BlockSpec((bn, F_IN), lambda i: (i, 0)),
            pl.BlockSpec((TOWERS, 4 * F_IN, 3 * G_T), lambda i: (0, 0, 0)),
            pl.BlockSpec((F_IN, F_IN), lambda i: (0, 0)),
            pl.BlockSpec((F_IN, F_IN), lambda i: (0, 0)),
            pl.BlockSpec((1, F_IN), lambda i: (0, 0)),
            pl.BlockSpec((1, F_IN), lambda i: (0, 0)),
        ],
        out_specs=[
            pl.BlockSpec((bn, F_IN), lambda i: (i, 0)),
            pl.BlockSpec((8, F_IN), lambda i: (0, 0)),
        ],
        out_shape=[
            jax.ShapeDtypeStruct((NPAD, F_IN), jnp.float32),
            jax.ShapeDtypeStruct((8, F_IN), jnp.float32),
        ],
    )(S, Q, MN, MX, C, cnt, x, Wagg, Wxf, WlT, bof, blf)


def _apply_kernel(y_ref, st_ref, g_ref, b_ref, o_ref):
    st = st_ref[...]
    m = st[0:1, :] * (1.0 / N_NODES)
    v = st[1:2, :] * (1.0 / N_NODES) - m * m
    o_ref[...] = jax.nn.relu(
        (y_ref[...] - m) / jnp.sqrt(v + 1e-5) * g_ref[...] + b_ref[...])


def _apply(y, st, g, b):
    bn, nbk = 1000, N_NODES // 1000
    return pl.pallas_call(
        _apply_kernel,
        grid=(nbk,),
        in_specs=[
            pl.BlockSpec((bn, F_IN), lambda i: (i, 0)),
            pl.BlockSpec((8, F_IN), lambda i: (0, 0)),
            pl.BlockSpec((1, F_IN), lambda i: (0, 0)),
            pl.BlockSpec((1, F_IN), lambda i: (0, 0)),
        ],
        out_specs=pl.BlockSpec((bn, F_IN), lambda i: (i, 0)),
        out_shape=jax.ShapeDtypeStruct((N_NODES, F_IN), jnp.float32),
    )(y, st, g.reshape(1, -1), b.reshape(1, -1))


def _prep_weights(Wp, bp, Wo, bo, Wl, bl):
    F = F_IN
    Wd = Wp[:, :, :F].transpose(2, 0, 1).reshape(F, TOWERS * F)
    Ws = Wp[:, :, F:].transpose(2, 0, 1).reshape(F, TOWERS * F)
    bpf = bp.reshape(1, TOWERS * F)
    Wxf = Wo[:, :, :F].transpose(2, 0, 1).reshape(F, TOWERS * G_T)
    Wr = Wo[:, :, F:5 * F].transpose(0, 2, 1)
    Wa = Wo[:, :, 5 * F:9 * F].transpose(0, 2, 1)
    Wb = Wo[:, :, 9 * F:].transpose(0, 2, 1)
    Wagg = jnp.concatenate([Wr, Wa, Wb], axis=2)
    return Wd, Ws, bpf, Wagg, Wxf, Wl.T, bo.reshape(1, -1), bl.reshape(1, -1)


def kernel(x, edge_index, pre1_W, pre1_b, post1_W, post1_b, lin1_W, lin1_b,
           bn1_g, bn1_b, pre2_W, pre2_b, post2_W, post2_b, lin2_W, lin2_b,
           bn2_g, bn2_b):
    src = edge_index[0].astype(jnp.int32)
    dst = edge_index[1].astype(jnp.int32)
    bl_lists, bl_cnts = _sc_prep(src, dst)

    Wd1, Ws1, bp1f, Wagg1, Wx1, Wl1T, bo1f, bl1f = _prep_weights(
        pre1_W, pre1_b, post1_W, post1_b, lin1_W, lin1_b)
    C1, B1 = _pre(x, Wd1, Ws1, bp1f)
    S1, Q1, MN1, MX1, CNT = _sc_layer_cnt(bl_lists, bl_cnts, B1)
    y1, st1 = _combine(S1, Q1, MN1, MX1, C1, CNT, x,
                       Wagg1, Wx1, Wl1T, bo1f, bl1f)
    h1 = _apply(y1, st1, bn1_g, bn1_b)

    Wd2, Ws2, bp2f, Wagg2, Wx2, Wl2T, bo2f, bl2f = _prep_weights(
        pre2_W, pre2_b, post2_W, post2_b, lin2_W, lin2_b)
    C2, B2 = _pre(h1, Wd2, Ws2, bp2f)
    S2, Q2, MN2, MX2 = _sc_layer_nocnt(bl_lists, bl_cnts, B2)
    y2, st2 = _combine(S2, Q2, MN2, MX2, C2, CNT, h1,
                       Wagg2, Wx2, Wl2T, bo2f, bl2f)
    h2 = _apply(y2, st2, bn2_g, bn2_b)
    return h2


# speedup vs baseline: 43.9188x; 1.1585x over previous
import functools
import math

import jax
import jax.numpy as jnp
import numpy as np
from jax import lax
from jax.experimental import pallas as pl
from jax.experimental.pallas import tpu as pltpu
from jax.experimental.pallas import tpu_sc as plsc

N_NODES = 10000
N_EDGES = 160000
F_IN = 128
TOWERS = 4
G_T = 32
_DEG = np.array([0, 0, 0, 0, 0, 0, 0, 0, 100, 200, 400, 600, 800, 1000, 1200,
                 1300, 1200, 1000, 800, 600, 400, 200, 100, 60, 40],
                dtype=np.float64)
AVG_LOG = float((np.log(np.arange(_DEG.shape[0]) + 1.0) * _DEG).sum() / _DEG.sum())

NSLICE = 32
EPS = N_EDGES // NSLICE
NRANGES = 64
NRANGE = 160
NPAD = NRANGES * NRANGE
BCAP = 160
BSTRIDE = 176
LCAP = NSLICE * BSTRIDE
KB = 64
NRP = 164
TRASH = 162
HI = jax.lax.Precision.HIGHEST

_mesh = plsc.VectorSubcoreMesh(core_axis_name="c", subcore_axis_name="s")


def _sc_prep(src, dst):
    BLROW = NRANGES * BSTRIDE

    @functools.partial(
        pl.kernel, mesh=_mesh,
        out_type=(jax.ShapeDtypeStruct((NSLICE * BLROW,), jnp.int32),
                  jax.ShapeDtypeStruct(((NRANGES + 1) * 512,), jnp.int32)),
        scratch_types=[pltpu.VMEM((EPS + 16,), jnp.int32),
                       pltpu.VMEM((EPS + 16,), jnp.int32),
                       pltpu.VMEM(((NRANGES + 1) * BSTRIDE,), jnp.int32),
                       pltpu.VMEM(((NRANGES + 1) * 16,), jnp.int32),
                       pltpu.SMEM((NRANGES + 1,), jnp.int32),
                       pltpu.SemaphoreType.DMA],
    )
    def k(src_hbm, dst_hbm, bl_hbm, cnts_hbm, sbuf, dbuf, bkt, cvec, cur, sem):
        w = lax.axis_index("s") * 2 + lax.axis_index("c")
        base = pl.multiple_of(w * EPS, 8)
        pltpu.sync_copy(src_hbm.at[pl.ds(base, EPS)], sbuf.at[pl.ds(0, EPS)])
        pltpu.sync_copy(dst_hbm.at[pl.ds(base, EPS)], dbuf.at[pl.ds(0, EPS)])

        def zc(i, _):
            cur[i] = 0
            return 0
        lax.fori_loop(0, NRANGES + 1, zc, 0)

        recip = np.float32((1.0 / NRANGE) * (1.0 + 2e-6))
        ngroups = (EPS + 15) // 16

        def grp(g, _):
            dv = dbuf[pl.ds(g * 16, 16)]
            sv = sbuf[pl.ds(g * 16, 16)]
            lv = (dv.astype(jnp.float32) * recip).astype(jnp.int32)
            valid = (g * 16 + lax.iota(jnp.int32, 16)) < EPS
            lv = jnp.where(valid, lv, NRANGES)
            pv = sv | ((dv - lv * NRANGE) << 16)
            pv = jnp.where(valid, pv, TRASH << 16)
            for j in range(16):
                l = lv[j]
                p = pv[j]
                c = cur[l]
                bkt[pl.ds(l * BSTRIDE + c, 16)] = jnp.full((16,), p, jnp.int32)
                cur[l] = jnp.minimum(c + 1, BCAP)
            return 0
        lax.fori_loop(0, ngroups, grp, 0)

        trash16 = jnp.full((16,), TRASH << 16, jnp.int32)

        def pad(l, _):
            c = cur[l]
            bkt[pl.ds(l * BSTRIDE + c, 16)] = trash16
            cur[l] = (c + 7) & (-8)
            return 0
        lax.fori_loop(0, NRANGES + 1, pad, 0)

        def cv(i, _):
            cvec[pl.ds(i * 16, 16)] = jnp.full((16,), cur[i], jnp.int32)
            return 0
        lax.fori_loop(0, NRANGES + 1, cv, 0)

        pltpu.sync_copy(bkt.at[pl.ds(0, BLROW)],
                        bl_hbm.at[pl.ds(pl.multiple_of(w * BLROW, 8), BLROW)])
        for l in range(NRANGES + 1):
            coff = pl.multiple_of(l * 512 + w * 16, 8)
            pltpu.async_copy(cvec.at[pl.ds(l * 16, 16)],
                             cnts_hbm.at[pl.ds(coff, 16)], sem)
        for l in range(NRANGES + 1):
            coff = pl.multiple_of(l * 512 + w * 16, 8)
            pltpu.make_async_copy(cvec.at[pl.ds(l * 16, 16)],
                                  cnts_hbm.at[pl.ds(coff, 16)], sem).wait()

    return k(src, dst)


def _make_sc_layer(compute_cnt):
    outs = [jax.ShapeDtypeStruct((TOWERS, NPAD, F_IN), jnp.float32)
            for _ in range(4)]
    if compute_cnt:
        outs.append(jax.ShapeDtypeStruct((NPAD, 16), jnp.float32))

    scratch = [pltpu.VMEM((LCAP,), jnp.int32),
               pltpu.VMEM((2, KB), jnp.int32),
               pltpu.VMEM((2, KB, F_IN), jnp.float32),
               pltpu.VMEM((NRP, F_IN), jnp.float32),
               pltpu.VMEM((NRP, F_IN), jnp.float32),
               pltpu.VMEM((NRP, F_IN), jnp.float32),
               pltpu.VMEM((NRP, F_IN), jnp.float32),
               pltpu.VMEM((512,), jnp.int32),
               pltpu.VMEM((NRP, 16), jnp.float32),
               pltpu.SemaphoreType.DMA]

    @functools.partial(pl.kernel, mesh=_mesh, out_type=tuple(outs),
                       scratch_types=scratch)
    def k(bl_hbm, cnts_hbm, tab_hbm, *rest):
        if compute_cnt:
            S_hbm, Q_hbm, MN_hbm, MX_hbm, CNT_hbm = rest[:5]
            scr = rest[5:]
        else:
            S_hbm, Q_hbm, MN_hbm, MX_hbm = rest[:4]
            scr = rest[4:]
        listv, idxb, rows, SU, SQ, MN, MX, ccache, cntacc, sem = scr
        w = lax.axis_index("s") * 2 + lax.axis_index("c")

        zeros16 = jnp.zeros((16,), jnp.float32)
        ones16 = jnp.ones((16,), jnp.float32)
        pinf16 = jnp.full((16,), float("inf"), jnp.float32)
        ninf16 = jnp.full((16,), -float("inf"), jnp.float32)
        trash16 = jnp.full((16,), TRASH << 16, jnp.int32)

        BLROW = NRANGES * BSTRIDE

        def round_body(r, _):
            l = w * 2 + r
            lo = pl.multiple_of(l * NRANGE, 8)
            pltpu.sync_copy(
                cnts_hbm.at[pl.ds(pl.multiple_of(l * 512, 8), 512)], ccache)

            def cp(sl, cur):
                boff = pl.multiple_of(sl * BLROW + l * BSTRIDE, 8)
                pltpu.sync_copy(
                    bl_hbm.at[pl.ds(boff, BSTRIDE)],
                    listv.at[pl.ds(pl.multiple_of(cur, 8), BSTRIDE)])
                crow = ccache[pl.ds(sl * 16, 16)]
                return cur + crow[0]
            cur = lax.fori_loop(0, NSLICE, cp, 0)

            def san(g, _):
                pv = listv[pl.ds(g * 16, 16)]
                vld = (g * 16 + lax.iota(jnp.int32, 16)) < cur
                listv[pl.ds(g * 16, 16)] = jnp.where(vld, pv, trash16)
                return 0
            lax.fori_loop(cur >> 4, LCAP // 16, san, 0)

            nb = (cur + (KB - 1)) >> 6

            def tower(t, _):
                tbase = t * N_NODES

                def zr(i, _):
                    for c8 in range(8):
                        sl16 = pl.ds(c8 * 16, 16)
                        SU[i, sl16] = zeros16
                        SQ[i, sl16] = zeros16
                        MN[i, sl16] = pinf16
                        MX[i, sl16] = ninf16
                    return 0
                lax.fori_loop(0, NRP, zr, 0)

                def start(b):
                    par = b & 1

                    def ig(g, _):
                        pv = listv[pl.ds(b * KB + g * 16, 16)]
                        idxb[par, pl.ds(g * 16, 16)] = (pv & 0xFFFF) + tbase
                        return 0
                    lax.fori_loop(0, KB // 16, ig, 0)
                    pltpu.async_copy(tab_hbm.at[idxb.at[par]],
                                     rows.at[par], sem)

                @pl.when(nb > 0)
                def _():
                    start(0)

                def bat(b, _):
                    par = b & 1
                    pltpu.make_async_copy(tab_hbm.at[idxb.at[par]],
                                          rows.at[par], sem).wait()

                    @pl.when(b + 1 < nb)
                    def _():
                        start(b + 1)

                    def sg_body(sg, _):
                        dlv = lax.shift_right_logical(
                            listv[pl.ds(b * KB + sg * 16, 16)], 16)
                        for j in range(16):
                            dl = dlv[j]
                            e = sg * 16 + j
                            for c8 in range(8):
                                sl16 = pl.ds(c8 * 16, 16)
                                bb = rows[par, e, sl16]
                                plsc.addupdate(SU.at[dl, sl16], bb)
                                plsc.addupdate(SQ.at[dl, sl16], bb * bb)
                                MN[dl, sl16] = jnp.minimum(MN[dl, sl16], bb)
                                MX[dl, sl16] = jnp.maximum(MX[dl, sl16], bb)
                        return 0
                    lax.fori_loop(0, KB // 16, sg_body, 0)
                    return 0
                lax.fori_loop(0, nb, bat, 0)

                pltpu.sync_copy(SU.at[pl.ds(0, NRANGE)],
                                S_hbm.at[t, pl.ds(lo, NRANGE)])
                pltpu.sync_copy(SQ.at[pl.ds(0, NRANGE)],
                                Q_hbm.at[t, pl.ds(lo, NRANGE)])
                pltpu.sync_copy(MN.at[pl.ds(0, NRANGE)],
                                MN_hbm.at[t, pl.ds(lo, NRANGE)])
                pltpu.sync_copy(MX.at[pl.ds(0, NRANGE)],
                                MX_hbm.at[t, pl.ds(lo, NRANGE)])
                return 0
            lax.fori_loop(0, TOWERS, tower, 0)

            if compute_cnt:
                def zc2(i, _):
                    cntacc[i, :] = zeros16
                    return 0
                lax.fori_loop(0, NRP, zc2, 0)

                def cg(g, _):
                    dlv = lax.shift_right_logical(
                        listv[pl.ds(g * 16, 16)], 16)
                    for j in range(16):
                        plsc.addupdate(cntacc.at[dlv[j], :], ones16)
                    return 0
                lax.fori_loop(0, (cur + 15) >> 4, cg, 0)
                pltpu.sync_copy(cntacc.at[pl.ds(0, NRANGE)],
                                CNT_hbm.at[pl.ds(lo, NRANGE)])
            return 0
        lax.fori_loop(0, 2, round_body, 0)

    return k


_sc_layer_cnt = _make_sc_layer(True)
_sc_layer_nocnt = _make_sc_layer(False)


def _pre_kernel(x_ref, wd_ref, ws_ref, b_ref, c_ref, bt_ref):
    xb = x_ref[...]
    c_ref[0] = jnp.dot(xb, wd_ref[...], precision=HI) + b_ref[...]
    bt_ref[...] = jnp.dot(xb, ws_ref[...], precision=HI)


def _pre(x, Wd, Ws, bpf):
    bn, nbk = 1000, N_NODES // 1000
    return pl.pallas_call(
        _pre_kernel,
        grid=(TOWERS, nbk),
        in_specs=[
            pl.BlockSpec((bn, F_IN), lambda t, i: (i, 0)),
            pl.BlockSpec((F_IN, F_IN), lambda t, i: (0, t)),
            pl.BlockSpec((F_IN, F_IN), lambda t, i: (0, t)),
            pl.BlockSpec((1, F_IN), lambda t, i: (0, t)),
        ],
        out_specs=[
            pl.BlockSpec((1, bn, F_IN), lambda t, i: (t, i, 0)),
            pl.BlockSpec((bn, F_IN), lambda t, i: (t * nbk + i, 0)),
        ],
        out_shape=[
            jax.ShapeDtypeStruct((TOWERS, N_NODES, F_IN), jnp.float32),
            jax.ShapeDtypeStruct((TOWERS * N_NODES, F_IN), jnp.float32),
        ],
    )(x, Wd, Ws, bpf)


def _combine_kernel(S_ref, Q_ref, MN_ref, MX_ref, C_ref, cnt_ref, x_ref,
                    Wagg_ref, Wx_ref, Wl_ref, bo_ref, bl_ref, y_ref, st_ref):
    cv = cnt_ref[...][:, 0]
    cntc = jnp.maximum(cv, 1.0)
    has = (cv > 0.0)[:, None]
    xb = x_ref[...]
    base = jnp.dot(xb, Wx_ref[...], precision=HI) + bo_ref[...]
    d = jnp.log(cntc + 1.0)
    s1 = (d / AVG_LOG)[:, None]
    s2 = (AVG_LOG / d)[:, None]
    inv = (1.0 / cntc)[:, None]
    parts = []
    for t in range(TOWERS):
        St = S_ref[t]
        Qt = Q_ref[t]
        Ct = C_ref[t]
        u = St * inv
        std = jnp.sqrt(jax.nn.relu(Qt * inv - u * u) + 1e-5)
        mean = jnp.where(has, Ct + u, 0.0)
        mn = jnp.where(has, Ct + MN_ref[t], 0.0)
        mx = jnp.where(has, Ct + MX_ref[t], 0.0)
        agg = jnp.concatenate([mean, mn, mx, std], axis=1)
        p = jnp.dot(agg, Wagg_ref[t], precision=HI)
        parts.append(p[:, 0:G_T] + s1 * p[:, G_T:2 * G_T]
                     + s2 * p[:, 2 * G_T:3 * G_T])
    post = jnp.concatenate(parts, axis=1) + base
    y = jnp.dot(post, Wl_ref[...], precision=HI) + bl_ref[...]
    y_ref[...] = y

    @pl.when(pl.program_id(0) == 0)
    def _():
        st_ref[...] = jnp.zeros((8, F_IN), jnp.float32)

    bn = y.shape[0]
    rid = (jax.lax.broadcasted_iota(jnp.int32, (bn, 1), 0)
           + pl.program_id(0) * bn)
    ym = jnp.where(rid < N_NODES, y, 0.0)
    upd = jnp.concatenate(
        [jnp.sum(ym, axis=0, keepdims=True),
         jnp.sum(ym * ym, axis=0, keepdims=True),
         jnp.zeros((6, F_IN), jnp.float32)], axis=0)
    st_ref[...] += upd


def _combine(S, Q, MN, MX, C, cnt, x, Wagg, Wxf, WlT, bof, blf):
    bn, nbk = 1280, NPAD // 1280
    full4 = pl.BlockSpec((TOWERS, bn, F_IN), lambda i: (0, i, 0))
    return pl.pallas_call(
        _combine_kernel,
        grid=(nbk,),
        in_specs=[
            full4, full4, full4, full4, full4,
            pl.BlockSpec((bn, 16), lambda i: (i, 0)),
            pl.BlockSpec((bn, F_IN), lambda i: (i, 0)),
            pl.BlockSpec((TOWERS, 4 * F_IN, 3 * G_T), lambda i: (0, 0, 0)),
            pl.BlockSpec((F_IN, F_IN), lambda i: (0, 0)),
            pl.BlockSpec((F_IN, F_IN), lambda i: (0, 0)),
            pl.BlockSpec((1, F_IN), lambda i: (0, 0)),
            pl.BlockSpec((1, F_IN), lambda i: (0, 0)),
        ],
        out_specs=[
            pl.BlockSpec((bn, F_IN), lambda i: (i, 0)),
            pl.BlockSpec((8, F_IN), lambda i: (0, 0)),
        ],
        out_shape=[
            jax.ShapeDtypeStruct((NPAD, F_IN), jnp.float32),
            jax.ShapeDtypeStruct((8, F_IN), jnp.float32),
        ],
    )(S, Q, MN, MX, C, cnt, x, Wagg, Wxf, WlT, bof, blf)


def _apply_kernel(y_ref, st_ref, g_ref, b_ref, o_ref):
    st = st_ref[...]
    m = st[0:1, :] * (1.0 / N_NODES)
    v = st[1:2, :] * (1.0 / N_NODES) - m * m
    o_ref[...] = jax.nn.relu(
        (y_ref[...] - m) / jnp.sqrt(v + 1e-5) * g_ref[...] + b_ref[...])


def _apply(y, st, g, b):
    bn, nbk = 1000, N_NODES // 1000
    return pl.pallas_call(
        _apply_kernel,
        grid=(nbk,),
        in_specs=[
            pl.BlockSpec((bn, F_IN), lambda i: (i, 0)),
            pl.BlockSpec((8, F_IN), lambda i: (0, 0)),
            pl.BlockSpec((1, F_IN), lambda i: (0, 0)),
            pl.BlockSpec((1, F_IN), lambda i: (0, 0)),
        ],
        out_specs=pl.BlockSpec((bn, F_IN), lambda i: (i, 0)),
        out_shape=jax.ShapeDtypeStruct((N_NODES, F_IN), jnp.float32),
    )(y, st, g.reshape(1, -1), b.reshape(1, -1))


def _prep_weights(Wp, bp, Wo, bo, Wl, bl):
    F = F_IN
    Wd = Wp[:, :, :F].transpose(2, 0, 1).reshape(F, TOWERS * F)
    Ws = Wp[:, :, F:].transpose(2, 0, 1).reshape(F, TOWERS * F)
    bpf = bp.reshape(1, TOWERS * F)
    Wxf = Wo[:, :, :F].transpose(2, 0, 1).reshape(F, TOWERS * G_T)
    Wr = Wo[:, :, F:5 * F].transpose(0, 2, 1)
    Wa = Wo[:, :, 5 * F:9 * F].transpose(0, 2, 1)
    Wb = Wo[:, :, 9 * F:].transpose(0, 2, 1)
    Wagg = jnp.concatenate([Wr, Wa, Wb], axis=2)
    return Wd, Ws, bpf, Wagg, Wxf, Wl.T, bo.reshape(1, -1), bl.reshape(1, -1)


def kernel(x, edge_index, pre1_W, pre1_b, post1_W, post1_b, lin1_W, lin1_b,
           bn1_g, bn1_b, pre2_W, pre2_b, post2_W, post2_b, lin2_W, lin2_b,
           bn2_g, bn2_b):
    src = edge_index[0].astype(jnp.int32)
    dst = edge_index[1].astype(jnp.int32)
    bl_lists, bl_cnts = _sc_prep(src, dst)

    Wd1, Ws1, bp1f, Wagg1, Wx1, Wl1T, bo1f, bl1f = _prep_weights(
        pre1_W, pre1_b, post1_W, post1_b, lin1_W, lin1_b)
    C1, B1 = _pre(x, Wd1, Ws1, bp1f)
    S1, Q1, MN1, MX1, CNT = _sc_layer_cnt(bl_lists, bl_cnts, B1)
    y1, st1 = _combine(S1, Q1, MN1, MX1, C1, CNT, x,
                       Wagg1, Wx1, Wl1T, bo1f, bl1f)
    h1 = _apply(y1, st1, bn1_g, bn1_b)

    Wd2, Ws2, bp2f, Wagg2, Wx2, Wl2T, bo2f, bl2f = _prep_weights(
        pre2_W, pre2_b, post2_W, post2_b, lin2_W, lin2_b)
    C2, B2 = _pre(h1, Wd2, Ws2, bp2f)
    S2, Q2, MN2, MX2 = _sc_layer_nocnt(bl_lists, bl_cnts, B2)
    y2, st2 = _combine(S2, Q2, MN2, MX2, C2, CNT, h1,
                       Wagg2, Wx2, Wl2T, bo2f, bl2f)
    h2 = _apply(y2, st2, bn2_g, bn2_b)
    return h2
